# baseline (speedup 1.0000x reference)
import functools

import jax
import jax.numpy as jnp
from jax import lax
from jax.experimental import pallas as pl
from jax.experimental.pallas import tpu as pltpu
from jax.experimental.pallas import tpu_sc as plsc

N = 10000
E = 320000
B = 128

NC = 2
NS = 16
NW = NC * NS
EPW = E // NW
CHUNK = 80
NCHUNK = EPW // CHUNK
RPT = N // NS
WBR = 624

_f32 = jnp.float32


def _sc_mesh():
    return plsc.VectorSubcoreMesh(core_axis_name="c", subcore_axis_name="s")


NBUF = 5


@functools.cache
def _sc_scatter(F):
    def body(y_hbm, src_hbm, dst_hbm, zero_hbm, out_hbm,
             srcall, dstall, *rest):
        msgbuf = rest[0:NBUF]
        acc_sh = rest[NBUF]
        gsem = rest[NBUF + 1:NBUF + 1 + NBUF]
        ssem = rest[2 * NBUF + 1:2 * NBUF + 1 + NBUF]
        c = lax.axis_index("c")
        s = lax.axis_index("s")
        w = c * NS + s

        @pl.when(s == 0)
        def _():
            pltpu.sync_copy(zero_hbm, acc_sh)

        pltpu.sync_copy(src_hbm.at[w], srcall)
        pltpu.sync_copy(dst_hbm.at[w], dstall)
        plsc.subcore_barrier()

        for b in range(NBUF):
            pltpu.async_copy(y_hbm.at[srcall.at[b]], msgbuf[b], gsem[b])

        LAG = 2

        def outer(j):
            for b in range(NBUF):
                i = j + b
                pltpu.make_async_copy(y_hbm.at[srcall.at[i]], msgbuf[b],
                                      gsem[b]).wait()
                pltpu.async_copy(msgbuf[b], acc_sh.at[dstall.at[i]],
                                 ssem[b], add=True)

                k = i - LAG
                bk = (b - LAG) % NBUF

                @pl.when((k >= 0) & (k + NBUF < NCHUNK))
                def _():
                    pltpu.make_async_copy(msgbuf[bk],
                                          acc_sh.at[dstall.at[i]],
                                          ssem[bk]).wait()
                    pltpu.async_copy(y_hbm.at[srcall.at[k + NBUF]], msgbuf[bk],
                                     gsem[bk])

        pl.loop(0, NCHUNK, step=NBUF)(outer)

        for b in range(NBUF):
            pltpu.make_async_copy(msgbuf[b],
                                  acc_sh.at[dstall.at[NCHUNK - NBUF + b]],
                                  ssem[b]).wait()

        plsc.subcore_barrier()
        pltpu.sync_copy(acc_sh.at[pl.ds(s * WBR, WBR)],
                        out_hbm.at[pl.ds(c * N + s * WBR, WBR)])

        @pl.when(s == NS - 1)
        def _():
            pltpu.sync_copy(acc_sh.at[pl.ds(NS * WBR, N - NS * WBR)],
                            out_hbm.at[pl.ds(c * N + NS * WBR, N - NS * WBR)])

    return pl.kernel(
        body,
        out_type=jax.ShapeDtypeStruct((NC * N, F), _f32),
        mesh=_sc_mesh(),
        compiler_params=pltpu.CompilerParams(use_tc_tiling_on_sc=False),
        scratch_types=[
            pltpu.VMEM((NCHUNK, CHUNK), jnp.int32),
            pltpu.VMEM((NCHUNK, CHUNK), jnp.int32),
        ] + [pltpu.VMEM((CHUNK, F), _f32) for _ in range(NBUF)] + [
            pltpu.VMEM_SHARED((N, F), _f32),
        ] + [pltpu.SemaphoreType.DMA for _ in range(2 * NBUF)],
    )


@functools.cache
def _sc_degree():
    def body(dst_hbm, ones_hbm, zero_hbm, out_hbm, onesbuf, dstall, deg_sh,
             sem):
        c = lax.axis_index("c")
        s = lax.axis_index("s")

        @pl.when(s == 0)
        def _():
            pltpu.sync_copy(zero_hbm, deg_sh)

        pltpu.sync_copy(ones_hbm, onesbuf)
        pltpu.sync_copy(dst_hbm.at[c * NS + s], dstall)
        plsc.subcore_barrier()

        def outer(j):
            for b in range(NBUF):
                pltpu.async_copy(onesbuf, deg_sh.at[dstall.at[j + b]], sem,
                                 add=True)
            for b in range(NBUF):
                pltpu.make_async_copy(onesbuf, deg_sh.at[dstall.at[j + b]],
                                      sem).wait()

        pl.loop(0, NCHUNK, step=NBUF)(outer)
        plsc.subcore_barrier()

        @pl.when(s == 0)
        def _():
            pltpu.sync_copy(deg_sh, out_hbm.at[pl.ds(c * N, N)])

    return pl.kernel(
        body,
        out_type=jax.ShapeDtypeStruct((NC * N,), _f32),
        mesh=_sc_mesh(),
        compiler_params=pltpu.CompilerParams(use_tc_tiling_on_sc=False),
        scratch_types=[
            pltpu.VMEM((CHUNK,), _f32),
            pltpu.VMEM((NCHUNK, CHUNK), jnp.int32),
            pltpu.VMEM_SHARED((N,), _f32),
            pltpu.SemaphoreType.DMA,
        ],
    )


def _tc_pre(degp, x, w0):
    def body(degp_ref, x_ref, w_ref, dinv_ref, y_ref):
        deg = degp_ref[0] + degp_ref[1] + 1.0
        dinv = lax.rsqrt(deg)
        dinv_ref[...] = dinv
        h = jnp.dot(x_ref[...], w_ref[...], preferred_element_type=_f32)
        y_ref[...] = h * dinv

    return pl.pallas_call(
        body,
        out_shape=(jax.ShapeDtypeStruct((N, 1), _f32),
                   jax.ShapeDtypeStruct((N, w0.shape[1]), _f32)),
    )(degp, x, w0)


@functools.cache
def _tc_mid(f_in, f_out, act):
    def body(accp_ref, y_ref, dinv_ref, b_ref, w_ref, ynew_ref):
        a = accp_ref[0] + accp_ref[1] + y_ref[...]
        z = a * dinv_ref[...] + b_ref[...]
        z = jnp.tanh(z) if act == "tanh" else jnp.maximum(z, 0.0)
        h = jnp.dot(z, w_ref[...], preferred_element_type=_f32)
        ynew_ref[...] = h * dinv_ref[...]

    return pl.pallas_call(
        body,
        out_shape=jax.ShapeDtypeStruct((N, f_out), _f32),
    )


RB = 400
NBLK = N // RB


def _tc_last(accp, y, dinv, batch_col, b3, wout, bout):
    F = 32

    def body(accp_ref, y_ref, dinv_ref, bat_ref, b_ref, wout_ref, bout_ref,
             out_ref, gmp_ref, ssum_ref, cnt_ref):
        i = pl.program_id(0)

        @pl.when(i == 0)
        def _():
            gmp_ref[...] = jnp.zeros_like(gmp_ref)
            ssum_ref[...] = jnp.zeros_like(ssum_ref)
            cnt_ref[...] = jnp.zeros_like(cnt_ref)

        a = accp_ref[0] + accp_ref[1] + y_ref[...]
        z = a * dinv_ref[...] + b_ref[...]
        z = jnp.maximum(z, 0.0)
        batc = bat_ref[...]
        seg_ids = lax.broadcasted_iota(jnp.int32, (1, B), 1)
        mask = (batc == seg_ids).astype(_f32)

        dn = (((0,), (0,)), ((), ()))
        ssum_ref[...] += lax.dot_general(mask, z, dn,
                                         preferred_element_type=_f32)
        cnt_ref[...] += lax.dot_general(mask, jnp.ones((RB, 1), _f32), dn,
                                        preferred_element_type=_f32)

        first = bat_ref[0, 0]
        last = bat_ref[RB - 1, 0]

        def seg(bidx, carry):
            m = batc == bidx
            zm = jnp.where(m, z, 0.0)
            vmax = jnp.max(zm, axis=0, keepdims=True)
            cur = gmp_ref[pl.ds(bidx, 1), :]
            gmp_ref[pl.ds(bidx, 1), :] = jnp.maximum(cur, vmax)
            return carry

        lax.fori_loop(first, last + 1, seg, 0)

        @pl.when(i == NBLK - 1)
        def _():
            cnt = cnt_ref[...]
            gm = jnp.where(cnt > 0.0, gmp_ref[...], -jnp.inf)
            gap = ssum_ref[...] / jnp.maximum(cnt, 1.0)
            w_max = wout_ref[pl.ds(0, F), :]
            w_avg = wout_ref[pl.ds(F, F), :]
            o = jnp.dot(gm, w_max, preferred_element_type=_f32)
            o += jnp.dot(gap, w_avg, preferred_element_type=_f32)
            out_ref[...] = o + bout_ref[...]

    return pl.pallas_call(
        body,
        grid=(NBLK,),
        in_specs=[
            pl.BlockSpec((2, RB, F), lambda i: (0, i, 0)),
            pl.BlockSpec((RB, F), lambda i: (i, 0)),
            pl.BlockSpec((RB, 1), lambda i: (i, 0)),
            pl.BlockSpec((RB, 1), lambda i: (i, 0)),
            pl.BlockSpec((1, F), lambda i: (0, 0)),
            pl.BlockSpec((2 * F, 10), lambda i: (0, 0)),
            pl.BlockSpec((1, 10), lambda i: (0, 0)),
        ],
        out_specs=pl.BlockSpec((B, 10), lambda i: (0, 0)),
        out_shape=jax.ShapeDtypeStruct((B, 10), _f32),
        scratch_shapes=[
            pltpu.VMEM((B, F), _f32),
            pltpu.VMEM((B, F), _f32),
            pltpu.VMEM((B, 1), _f32),
        ],
    )(accp, y, dinv, batch_col, b3, wout, bout)


def kernel(x, edge_index, batch_index, W0, b0, W1, b1, W2, b2, W3, b3,
           Wout, bout):
    src = edge_index[0].reshape(NW, NCHUNK, CHUNK)
    dst = edge_index[1].reshape(NW, NCHUNK, CHUNK)
    zeros64 = jnp.zeros((N, 64), _f32)
    zeros32 = jnp.zeros((N, 32), _f32)
    zeros1 = jnp.zeros((N,), _f32)
    ones_chunk = jnp.ones((CHUNK,), _f32)
    batch_col = batch_index.reshape(N, 1)

    degp = _sc_degree()(dst, ones_chunk, zeros1).reshape(NC, N, 1)
    dinv, y0 = _tc_pre(degp, x, W0)

    acc0 = _sc_scatter(64)(y0, src, dst, zeros64).reshape(NC, N, 64)
    y1 = _tc_mid(64, 64, "tanh")(acc0, y0, dinv, b0.reshape(1, -1), W1)

    acc1 = _sc_scatter(64)(y1, src, dst, zeros64).reshape(NC, N, 64)
    y2 = _tc_mid(64, 32, "relu")(acc1, y1, dinv, b1.reshape(1, -1), W2)

    acc2 = _sc_scatter(32)(y2, src, dst, zeros32).reshape(NC, N, 32)
    y3 = _tc_mid(32, 32, "relu")(acc2, y2, dinv, b2.reshape(1, -1), W3)

    acc3 = _sc_scatter(32)(y3, src, dst, zeros32).reshape(NC, N, 32)
    out = _tc_last(acc3, y3, dinv, batch_col, b3.reshape(1, -1), Wout,
                   bout.reshape(1, -1))
    return out


# speedup vs baseline: 1.0599x; 1.0599x over previous
import functools

import jax
import jax.numpy as jnp
from jax import lax
from jax.experimental import pallas as pl
from jax.experimental.pallas import tpu as pltpu
from jax.experimental.pallas import tpu_sc as plsc

N = 10000
E = 320000
B = 128

NC = 2
NS = 16
NW = NC * NS
EPW = E // NW
CHUNK = 80
NCHUNK = EPW // CHUNK
RPT = N // NS
WBR = 624

_f32 = jnp.float32


def _sc_mesh():
    return plsc.VectorSubcoreMesh(core_axis_name="c", subcore_axis_name="s")


NBUF = 5


@functools.cache
def _sc_scatter(F):
    def body(y_hbm, src_hbm, dst_hbm, zero_hbm, out_hbm,
             srcall, dstall, *rest):
        msgbuf = rest[0:NBUF]
        acc_sh = rest[NBUF]
        gsem = rest[NBUF + 1:NBUF + 1 + NBUF]
        ssem = rest[2 * NBUF + 1:2 * NBUF + 1 + NBUF]
        c = lax.axis_index("c")
        s = lax.axis_index("s")
        w = c * NS + s

        pltpu.sync_copy(zero_hbm.at[pl.ds(s * WBR, WBR)],
                        acc_sh.at[pl.ds(s * WBR, WBR)])

        @pl.when(s == NS - 1)
        def _():
            pltpu.sync_copy(zero_hbm.at[pl.ds(NS * WBR, N - NS * WBR)],
                            acc_sh.at[pl.ds(NS * WBR, N - NS * WBR)])

        pltpu.sync_copy(src_hbm.at[w], srcall)
        pltpu.sync_copy(dst_hbm.at[w], dstall)
        plsc.subcore_barrier()

        for b in range(NBUF):
            pltpu.async_copy(y_hbm.at[srcall.at[b]], msgbuf[b], gsem[b])

        def outer(j):
            for b in range(NBUF):
                i = j + b
                pltpu.make_async_copy(y_hbm.at[srcall.at[i]], msgbuf[b],
                                      gsem[b]).wait()
                pltpu.async_copy(msgbuf[b], acc_sh.at[dstall.at[i]],
                                 ssem[b], add=True)

                @pl.when(i + NBUF < NCHUNK)
                def _():
                    pltpu.make_async_copy(msgbuf[b],
                                          acc_sh.at[dstall.at[i]],
                                          ssem[b]).wait()
                    pltpu.async_copy(y_hbm.at[srcall.at[i + NBUF]], msgbuf[b],
                                     gsem[b])

        pl.loop(0, NCHUNK, step=NBUF)(outer)

        for b in range(NBUF):
            pltpu.make_async_copy(msgbuf[b],
                                  acc_sh.at[dstall.at[NCHUNK - NBUF + b]],
                                  ssem[b]).wait()

        plsc.subcore_barrier()
        pltpu.sync_copy(acc_sh.at[pl.ds(s * WBR, WBR)],
                        out_hbm.at[pl.ds(c * N + s * WBR, WBR)])

        @pl.when(s == NS - 1)
        def _():
            pltpu.sync_copy(acc_sh.at[pl.ds(NS * WBR, N - NS * WBR)],
                            out_hbm.at[pl.ds(c * N + NS * WBR, N - NS * WBR)])

    return pl.kernel(
        body,
        out_type=jax.ShapeDtypeStruct((NC * N, F), _f32),
        mesh=_sc_mesh(),
        compiler_params=pltpu.CompilerParams(use_tc_tiling_on_sc=False),
        scratch_types=[
            pltpu.VMEM((NCHUNK, CHUNK), jnp.int32),
            pltpu.VMEM((NCHUNK, CHUNK), jnp.int32),
        ] + [pltpu.VMEM((CHUNK, F), _f32) for _ in range(NBUF)] + [
            pltpu.VMEM_SHARED((N, F), _f32),
        ] + [pltpu.SemaphoreType.DMA for _ in range(2 * NBUF)],
    )


@functools.cache
def _sc_degree():
    def body(dst_hbm, ones_hbm, zero_hbm, out_hbm, onesbuf, dstall, deg_sh,
             sem):
        c = lax.axis_index("c")
        s = lax.axis_index("s")

        @pl.when(s == 0)
        def _():
            pltpu.sync_copy(zero_hbm, deg_sh)

        pltpu.sync_copy(ones_hbm, onesbuf)
        pltpu.sync_copy(dst_hbm.at[c * NS + s], dstall)
        plsc.subcore_barrier()

        def outer(j):
            for b in range(NBUF):
                pltpu.async_copy(onesbuf, deg_sh.at[dstall.at[j + b]], sem,
                                 add=True)
            for b in range(NBUF):
                pltpu.make_async_copy(onesbuf, deg_sh.at[dstall.at[j + b]],
                                      sem).wait()

        pl.loop(0, NCHUNK, step=NBUF)(outer)
        plsc.subcore_barrier()

        @pl.when(s == 0)
        def _():
            pltpu.sync_copy(deg_sh, out_hbm.at[pl.ds(c * N, N)])

    return pl.kernel(
        body,
        out_type=jax.ShapeDtypeStruct((NC * N,), _f32),
        mesh=_sc_mesh(),
        compiler_params=pltpu.CompilerParams(use_tc_tiling_on_sc=False),
        scratch_types=[
            pltpu.VMEM((CHUNK,), _f32),
            pltpu.VMEM((NCHUNK, CHUNK), jnp.int32),
            pltpu.VMEM_SHARED((N,), _f32),
            pltpu.SemaphoreType.DMA,
        ],
    )


def _tc_pre(degp, x, w0):
    def body(degp_ref, x_ref, w_ref, dinv_ref, y_ref):
        deg = degp_ref[0] + degp_ref[1] + 1.0
        dinv = lax.rsqrt(deg)
        dinv_ref[...] = dinv
        h = jnp.dot(x_ref[...], w_ref[...], preferred_element_type=_f32)
        y_ref[...] = h * dinv

    return pl.pallas_call(
        body,
        out_shape=(jax.ShapeDtypeStruct((N, 1), _f32),
                   jax.ShapeDtypeStruct((N, w0.shape[1]), _f32)),
    )(degp, x, w0)


@functools.cache
def _tc_mid(f_in, f_out, act):
    def body(accp_ref, y_ref, dinv_ref, b_ref, w_ref, ynew_ref):
        a = accp_ref[0] + accp_ref[1] + y_ref[...]
        z = a * dinv_ref[...] + b_ref[...]
        z = jnp.tanh(z) if act == "tanh" else jnp.maximum(z, 0.0)
        h = jnp.dot(z, w_ref[...], preferred_element_type=_f32)
        ynew_ref[...] = h * dinv_ref[...]

    return pl.pallas_call(
        body,
        out_shape=jax.ShapeDtypeStruct((N, f_out), _f32),
    )


RB = 1000
NBLK = N // RB


def _tc_last(accp, y, dinv, batch_col, b3, wout, bout):
    F = 32

    def body(accp_ref, y_ref, dinv_ref, bat_ref, b_ref, wout_ref, bout_ref,
             out_ref, gmp_ref, ssum_ref, cnt_ref):
        i = pl.program_id(0)

        @pl.when(i == 0)
        def _():
            gmp_ref[...] = jnp.zeros_like(gmp_ref)
            ssum_ref[...] = jnp.zeros_like(ssum_ref)
            cnt_ref[...] = jnp.zeros_like(cnt_ref)

        a = accp_ref[0] + accp_ref[1] + y_ref[...]
        z = a * dinv_ref[...] + b_ref[...]
        z = jnp.maximum(z, 0.0)
        batc = bat_ref[...]
        seg_ids = lax.broadcasted_iota(jnp.int32, (1, B), 1)
        mask = (batc == seg_ids).astype(_f32)

        dn = (((0,), (0,)), ((), ()))
        ssum_ref[...] += lax.dot_general(mask, z, dn,
                                         preferred_element_type=_f32)
        cnt_ref[...] += lax.dot_general(mask, jnp.ones((RB, 1), _f32), dn,
                                        preferred_element_type=_f32)

        first = bat_ref[0, 0]
        last = bat_ref[RB - 1, 0]

        def seg(bidx, carry):
            m = batc == bidx
            zm = jnp.where(m, z, 0.0)
            vmax = jnp.max(zm, axis=0, keepdims=True)
            cur = gmp_ref[pl.ds(bidx, 1), :]
            gmp_ref[pl.ds(bidx, 1), :] = jnp.maximum(cur, vmax)
            return carry

        lax.fori_loop(first, last + 1, seg, 0)

        @pl.when(i == NBLK - 1)
        def _():
            cnt = cnt_ref[...]
            gm = jnp.where(cnt > 0.0, gmp_ref[...], -jnp.inf)
            gap = ssum_ref[...] / jnp.maximum(cnt, 1.0)
            w_max = wout_ref[pl.ds(0, F), :]
            w_avg = wout_ref[pl.ds(F, F), :]
            o = jnp.dot(gm, w_max, preferred_element_type=_f32)
            o += jnp.dot(gap, w_avg, preferred_element_type=_f32)
            out_ref[...] = o + bout_ref[...]

    return pl.pallas_call(
        body,
        grid=(NBLK,),
        in_specs=[
            pl.BlockSpec((2, RB, F), lambda i: (0, i, 0)),
            pl.BlockSpec((RB, F), lambda i: (i, 0)),
            pl.BlockSpec((RB, 1), lambda i: (i, 0)),
            pl.BlockSpec((RB, 1), lambda i: (i, 0)),
            pl.BlockSpec((1, F), lambda i: (0, 0)),
            pl.BlockSpec((2 * F, 10), lambda i: (0, 0)),
            pl.BlockSpec((1, 10), lambda i: (0, 0)),
        ],
        out_specs=pl.BlockSpec((B, 10), lambda i: (0, 0)),
        out_shape=jax.ShapeDtypeStruct((B, 10), _f32),
        scratch_shapes=[
            pltpu.VMEM((B, F), _f32),
            pltpu.VMEM((B, F), _f32),
            pltpu.VMEM((B, 1), _f32),
        ],
    )(accp, y, dinv, batch_col, b3, wout, bout)


def kernel(x, edge_index, batch_index, W0, b0, W1, b1, W2, b2, W3, b3,
           Wout, bout):
    src = edge_index[0].reshape(NW, NCHUNK, CHUNK)
    dst = edge_index[1].reshape(NW, NCHUNK, CHUNK)
    zeros64 = jnp.zeros((N, 64), _f32)
    zeros32 = jnp.zeros((N, 32), _f32)
    zeros1 = jnp.zeros((N,), _f32)
    ones_chunk = jnp.ones((CHUNK,), _f32)
    batch_col = batch_index.reshape(N, 1)

    degp = _sc_degree()(dst, ones_chunk, zeros1).reshape(NC, N, 1)
    dinv, y0 = _tc_pre(degp, x, W0)

    acc0 = _sc_scatter(64)(y0, src, dst, zeros64).reshape(NC, N, 64)
    y1 = _tc_mid(64, 64, "tanh")(acc0, y0, dinv, b0.reshape(1, -1), W1)

    acc1 = _sc_scatter(64)(y1, src, dst, zeros64).reshape(NC, N, 64)
    y2 = _tc_mid(64, 32, "relu")(acc1, y1, dinv, b1.reshape(1, -1), W2)

    acc2 = _sc_scatter(32)(y2, src, dst, zeros32).reshape(NC, N, 32)
    y3 = _tc_mid(32, 32, "relu")(acc2, y2, dinv, b2.reshape(1, -1), W3)

    acc3 = _sc_scatter(32)(y3, src, dst, zeros32).reshape(NC, N, 32)
    out = _tc_last(acc3, y3, dinv, batch_col, b3.reshape(1, -1), Wout,
                   bout.reshape(1, -1))
    return out


# speedup vs baseline: 1.0721x; 1.0115x over previous
import functools

import jax
import jax.numpy as jnp
from jax import lax
from jax.experimental import pallas as pl
from jax.experimental.pallas import tpu as pltpu
from jax.experimental.pallas import tpu_sc as plsc

N = 10000
E = 320000
B = 128

NC = 2
NS = 16
NW = NC * NS
EPW = E // NW
CHUNK = 80
NCHUNK = EPW // CHUNK
RPT = N // NS
WBR = 624

_f32 = jnp.float32


def _sc_mesh():
    return plsc.VectorSubcoreMesh(core_axis_name="c", subcore_axis_name="s")


NBUF = 5


@functools.cache
def _sc_scatter(F):
    def body(y_hbm, src_hbm, dst_hbm, zero_hbm, out_hbm,
             srcall, dstall, *rest):
        msgbuf = rest[0:NBUF]
        acc_sh = rest[NBUF]
        gsem = rest[NBUF + 1:NBUF + 1 + NBUF]
        ssem = rest[2 * NBUF + 1:2 * NBUF + 1 + NBUF]
        c = lax.axis_index("c")
        s = lax.axis_index("s")
        w = c * NS + s

        pltpu.sync_copy(zero_hbm.at[pl.ds(s * WBR, WBR)],
                        acc_sh.at[pl.ds(s * WBR, WBR)])

        @pl.when(s == NS - 1)
        def _():
            pltpu.sync_copy(zero_hbm.at[pl.ds(NS * WBR, N - NS * WBR)],
                            acc_sh.at[pl.ds(NS * WBR, N - NS * WBR)])

        pltpu.sync_copy(src_hbm.at[w], srcall)
        pltpu.sync_copy(dst_hbm.at[w], dstall)
        plsc.subcore_barrier()

        for b in range(NBUF):
            pltpu.async_copy(y_hbm.at[srcall.at[b]], msgbuf[b], gsem[b])

        def outer(j):
            for b in range(NBUF):
                i = j + b
                pltpu.make_async_copy(y_hbm.at[srcall.at[i]], msgbuf[b],
                                      gsem[b]).wait()
                pltpu.async_copy(msgbuf[b], acc_sh.at[dstall.at[i]],
                                 ssem[b], add=True)

                @pl.when(i + NBUF < NCHUNK)
                def _():
                    pltpu.make_async_copy(msgbuf[b],
                                          acc_sh.at[dstall.at[i]],
                                          ssem[b]).wait()
                    pltpu.async_copy(y_hbm.at[srcall.at[i + NBUF]], msgbuf[b],
                                     gsem[b])

        pl.loop(0, NCHUNK, step=NBUF)(outer)

        for b in range(NBUF):
            pltpu.make_async_copy(msgbuf[b],
                                  acc_sh.at[dstall.at[NCHUNK - NBUF + b]],
                                  ssem[b]).wait()

        plsc.subcore_barrier()
        pltpu.sync_copy(acc_sh.at[pl.ds(s * WBR, WBR)],
                        out_hbm.at[pl.ds(c * N + s * WBR, WBR)])

        @pl.when(s == NS - 1)
        def _():
            pltpu.sync_copy(acc_sh.at[pl.ds(NS * WBR, N - NS * WBR)],
                            out_hbm.at[pl.ds(c * N + NS * WBR, N - NS * WBR)])

    return pl.kernel(
        body,
        out_type=jax.ShapeDtypeStruct((NC * N, F), _f32),
        mesh=_sc_mesh(),
        compiler_params=pltpu.CompilerParams(use_tc_tiling_on_sc=False),
        scratch_types=[
            pltpu.VMEM((NCHUNK, CHUNK), jnp.int32),
            pltpu.VMEM((NCHUNK, CHUNK), jnp.int32),
        ] + [pltpu.VMEM((CHUNK, F), _f32) for _ in range(NBUF)] + [
            pltpu.VMEM_SHARED((N, F), _f32),
        ] + [pltpu.SemaphoreType.DMA for _ in range(2 * NBUF)],
    )


@functools.cache
def _sc_degree():
    def body(dst_hbm, ones_hbm, zero_hbm, out_hbm, onesbuf, dstall, deg_sh,
             sem):
        c = lax.axis_index("c")
        s = lax.axis_index("s")

        @pl.when(s == 0)
        def _():
            pltpu.sync_copy(zero_hbm, deg_sh)

        pltpu.sync_copy(ones_hbm, onesbuf)
        pltpu.sync_copy(dst_hbm.at[c * NS + s], dstall)
        plsc.subcore_barrier()

        def outer(j):
            for b in range(NBUF):
                pltpu.async_copy(onesbuf, deg_sh.at[dstall.at[j + b]], sem,
                                 add=True)
            for b in range(NBUF):
                pltpu.make_async_copy(onesbuf, deg_sh.at[dstall.at[j + b]],
                                      sem).wait()

        pl.loop(0, NCHUNK, step=NBUF)(outer)
        plsc.subcore_barrier()

        @pl.when(s == 0)
        def _():
            pltpu.sync_copy(deg_sh, out_hbm.at[pl.ds(c * N, N)])

    return pl.kernel(
        body,
        out_type=jax.ShapeDtypeStruct((NC * N,), _f32),
        mesh=_sc_mesh(),
        compiler_params=pltpu.CompilerParams(use_tc_tiling_on_sc=False),
        scratch_types=[
            pltpu.VMEM((CHUNK,), _f32),
            pltpu.VMEM((NCHUNK, CHUNK), jnp.int32),
            pltpu.VMEM_SHARED((N,), _f32),
            pltpu.SemaphoreType.DMA,
        ],
    )


def _tc_pre(degp, x, w0):
    def body(degp_ref, x_ref, w_ref, dinv_ref, y_ref):
        deg = degp_ref[0] + degp_ref[1] + 1.0
        dinv = lax.rsqrt(deg)
        dinv_ref[...] = dinv
        h = jnp.dot(x_ref[...], w_ref[...], preferred_element_type=_f32)
        y_ref[...] = h * dinv

    return pl.pallas_call(
        body,
        out_shape=(jax.ShapeDtypeStruct((N, 1), _f32),
                   jax.ShapeDtypeStruct((N, w0.shape[1]), _f32)),
    )(degp, x, w0)


@functools.cache
def _tc_mid(f_in, f_out, act):
    def body(accp_ref, y_ref, dinv_ref, b_ref, w_ref, ynew_ref):
        a = accp_ref[0] + accp_ref[1] + y_ref[...]
        z = a * dinv_ref[...] + b_ref[...]
        z = jnp.tanh(z) if act == "tanh" else jnp.maximum(z, 0.0)
        h = jnp.dot(z, w_ref[...], preferred_element_type=_f32)
        ynew_ref[...] = h * dinv_ref[...]

    return pl.pallas_call(
        body,
        out_shape=jax.ShapeDtypeStruct((N, f_out), _f32),
    )


RB = 400
NBLK = N // RB


def _tc_last(accp, y, dinv, batch_col, b3, wout, bout):
    F = 32

    def body(accp_ref, y_ref, dinv_ref, bat_ref, b_ref, wout_ref, bout_ref,
             out_ref, gmp_ref, ssum_ref, cnt_ref):
        i = pl.program_id(0)

        @pl.when(i == 0)
        def _():
            gmp_ref[...] = jnp.zeros_like(gmp_ref)
            ssum_ref[...] = jnp.zeros_like(ssum_ref)
            cnt_ref[...] = jnp.zeros_like(cnt_ref)

        a = accp_ref[0] + accp_ref[1] + y_ref[...]
        z = a * dinv_ref[...] + b_ref[...]
        z = jnp.maximum(z, 0.0)
        batc = bat_ref[...]
        seg_ids = lax.broadcasted_iota(jnp.int32, (1, B), 1)
        mask = (batc == seg_ids).astype(_f32)

        dn = (((0,), (0,)), ((), ()))
        ssum_ref[...] += lax.dot_general(mask, z, dn,
                                         preferred_element_type=_f32)
        cnt_ref[...] += lax.dot_general(mask, jnp.ones((RB, 1), _f32), dn,
                                        preferred_element_type=_f32)

        first = bat_ref[0, 0]
        last = bat_ref[RB - 1, 0]

        def seg(bidx, carry):
            m = batc == bidx
            zm = jnp.where(m, z, 0.0)
            vmax = jnp.max(zm, axis=0, keepdims=True)
            cur = gmp_ref[pl.ds(bidx, 1), :]
            gmp_ref[pl.ds(bidx, 1), :] = jnp.maximum(cur, vmax)
            return carry

        lax.fori_loop(first, last + 1, seg, 0)

        @pl.when(i == NBLK - 1)
        def _():
            cnt = cnt_ref[...]
            gm = jnp.where(cnt > 0.0, gmp_ref[...], -jnp.inf)
            gap = ssum_ref[...] / jnp.maximum(cnt, 1.0)
            w_max = wout_ref[pl.ds(0, F), :]
            w_avg = wout_ref[pl.ds(F, F), :]
            o = jnp.dot(gm, w_max, preferred_element_type=_f32)
            o += jnp.dot(gap, w_avg, preferred_element_type=_f32)
            out_ref[...] = o + bout_ref[...]

    return pl.pallas_call(
        body,
        grid=(NBLK,),
        in_specs=[
            pl.BlockSpec((2, RB, F), lambda i: (0, i, 0)),
            pl.BlockSpec((RB, F), lambda i: (i, 0)),
            pl.BlockSpec((RB, 1), lambda i: (i, 0)),
            pl.BlockSpec((RB, 1), lambda i: (i, 0)),
            pl.BlockSpec((1, F), lambda i: (0, 0)),
            pl.BlockSpec((2 * F, 10), lambda i: (0, 0)),
            pl.BlockSpec((1, 10), lambda i: (0, 0)),
        ],
        out_specs=pl.BlockSpec((B, 10), lambda i: (0, 0)),
        out_shape=jax.ShapeDtypeStruct((B, 10), _f32),
        scratch_shapes=[
            pltpu.VMEM((B, F), _f32),
            pltpu.VMEM((B, F), _f32),
            pltpu.VMEM((B, 1), _f32),
        ],
    )(accp, y, dinv, batch_col, b3, wout, bout)


def kernel(x, edge_index, batch_index, W0, b0, W1, b1, W2, b2, W3, b3,
           Wout, bout):
    src = edge_index[0].reshape(NW, NCHUNK, CHUNK)
    dst = edge_index[1].reshape(NW, NCHUNK, CHUNK)
    zeros64 = jnp.zeros((N, 64), _f32)
    zeros32 = jnp.zeros((N, 32), _f32)
    zeros1 = jnp.zeros((N,), _f32)
    ones_chunk = jnp.ones((CHUNK,), _f32)
    batch_col = batch_index.reshape(N, 1)

    degp = _sc_degree()(dst, ones_chunk, zeros1).reshape(NC, N, 1)
    dinv, y0 = _tc_pre(degp, x, W0)

    acc0 = _sc_scatter(64)(y0, src, dst, zeros64).reshape(NC, N, 64)
    y1 = _tc_mid(64, 64, "tanh")(acc0, y0, dinv, b0.reshape(1, -1), W1)

    acc1 = _sc_scatter(64)(y1, src, dst, zeros64).reshape(NC, N, 64)
    y2 = _tc_mid(64, 32, "relu")(acc1, y1, dinv, b1.reshape(1, -1), W2)

    acc2 = _sc_scatter(32)(y2, src, dst, zeros32).reshape(NC, N, 32)
    y3 = _tc_mid(32, 32, "relu")(acc2, y2, dinv, b2.reshape(1, -1), W3)

    acc3 = _sc_scatter(32)(y3, src, dst, zeros32).reshape(NC, N, 32)
    out = _tc_last(acc3, y3, dinv, batch_col, b3.reshape(1, -1), Wout,
                   bout.reshape(1, -1))
    return out


# speedup vs baseline: 1.0933x; 1.0198x over previous
import functools

import jax
import jax.numpy as jnp
from jax import lax
from jax.experimental import pallas as pl
from jax.experimental.pallas import tpu as pltpu
from jax.experimental.pallas import tpu_sc as plsc

N = 10000
E = 320000
B = 128

NC = 2
NS = 16
NW = NC * NS
EPW = E // NW
CHUNK = 80
NCHUNK = EPW // CHUNK
RPT = N // NS
WBR = 624

_f32 = jnp.float32


def _sc_mesh():
    return plsc.VectorSubcoreMesh(core_axis_name="c", subcore_axis_name="s")


NBUF = 5


@functools.cache
def _sc_scatter(F):
    def body(y_hbm, edge_hbm, zero_hbm, out_hbm,
             srcall, dstall, *rest):
        msgbuf = rest[0:NBUF]
        acc_sh = rest[NBUF]
        gsem = rest[NBUF + 1:NBUF + 1 + NBUF]
        ssem = rest[2 * NBUF + 1:2 * NBUF + 1 + NBUF]
        c = lax.axis_index("c")
        s = lax.axis_index("s")
        w = c * NS + s

        pltpu.sync_copy(zero_hbm.at[pl.ds(s * WBR, WBR)],
                        acc_sh.at[pl.ds(s * WBR, WBR)])

        @pl.when(s == NS - 1)
        def _():
            pltpu.sync_copy(zero_hbm.at[pl.ds(NS * WBR, N - NS * WBR)],
                            acc_sh.at[pl.ds(NS * WBR, N - NS * WBR)])

        pltpu.sync_copy(edge_hbm.at[0, w], srcall)
        pltpu.sync_copy(edge_hbm.at[1, w], dstall)
        plsc.subcore_barrier()

        for b in range(NBUF):
            pltpu.async_copy(y_hbm.at[srcall.at[b]], msgbuf[b], gsem[b])

        def outer(j):
            for b in range(NBUF):
                i = j + b
                pltpu.make_async_copy(y_hbm.at[srcall.at[i]], msgbuf[b],
                                      gsem[b]).wait()
                pltpu.async_copy(msgbuf[b], acc_sh.at[dstall.at[i]],
                                 ssem[b], add=True)

                @pl.when(i + NBUF < NCHUNK)
                def _():
                    pltpu.make_async_copy(msgbuf[b],
                                          acc_sh.at[dstall.at[i]],
                                          ssem[b]).wait()
                    pltpu.async_copy(y_hbm.at[srcall.at[i + NBUF]], msgbuf[b],
                                     gsem[b])

        pl.loop(0, NCHUNK, step=NBUF)(outer)

        for b in range(NBUF):
            pltpu.make_async_copy(msgbuf[b],
                                  acc_sh.at[dstall.at[NCHUNK - NBUF + b]],
                                  ssem[b]).wait()

        plsc.subcore_barrier()
        pltpu.sync_copy(acc_sh.at[pl.ds(s * WBR, WBR)],
                        out_hbm.at[pl.ds(c * N + s * WBR, WBR)])

        @pl.when(s == NS - 1)
        def _():
            pltpu.sync_copy(acc_sh.at[pl.ds(NS * WBR, N - NS * WBR)],
                            out_hbm.at[pl.ds(c * N + NS * WBR, N - NS * WBR)])

    return pl.kernel(
        body,
        out_type=jax.ShapeDtypeStruct((NC * N, F), _f32),
        mesh=_sc_mesh(),
        compiler_params=pltpu.CompilerParams(use_tc_tiling_on_sc=False),
        scratch_types=[
            pltpu.VMEM((NCHUNK, CHUNK), jnp.int32),
            pltpu.VMEM((NCHUNK, CHUNK), jnp.int32),
        ] + [pltpu.VMEM((CHUNK, F), _f32) for _ in range(NBUF)] + [
            pltpu.VMEM_SHARED((N, F), _f32),
        ] + [pltpu.SemaphoreType.DMA for _ in range(2 * NBUF)],
    )


@functools.cache
def _sc_degree():
    def body(edge_hbm, ones_hbm, zero_hbm, out_hbm, onesbuf, dstall, deg_sh,
             sem):
        c = lax.axis_index("c")
        s = lax.axis_index("s")

        pltpu.sync_copy(zero_hbm.at[pl.ds(s * WBR, WBR)],
                        deg_sh.at[pl.ds(s * WBR, WBR)])

        @pl.when(s == NS - 1)
        def _():
            pltpu.sync_copy(zero_hbm.at[pl.ds(NS * WBR, N - NS * WBR)],
                            deg_sh.at[pl.ds(NS * WBR, N - NS * WBR)])

        pltpu.sync_copy(ones_hbm, onesbuf)
        pltpu.sync_copy(edge_hbm.at[1, c * NS + s], dstall)
        plsc.subcore_barrier()

        def outer(j):
            for b in range(NBUF):
                pltpu.async_copy(onesbuf, deg_sh.at[dstall.at[j + b]], sem,
                                 add=True)
            for b in range(NBUF):
                pltpu.make_async_copy(onesbuf, deg_sh.at[dstall.at[j + b]],
                                      sem).wait()

        pl.loop(0, NCHUNK, step=NBUF)(outer)
        plsc.subcore_barrier()
        pltpu.sync_copy(deg_sh.at[pl.ds(s * WBR, WBR)],
                        out_hbm.at[pl.ds(c * N + s * WBR, WBR)])

        @pl.when(s == NS - 1)
        def _():
            pltpu.sync_copy(deg_sh.at[pl.ds(NS * WBR, N - NS * WBR)],
                            out_hbm.at[pl.ds(c * N + NS * WBR, N - NS * WBR)])

    return pl.kernel(
        body,
        out_type=jax.ShapeDtypeStruct((NC * N, 8), _f32),
        mesh=_sc_mesh(),
        compiler_params=pltpu.CompilerParams(use_tc_tiling_on_sc=False),
        scratch_types=[
            pltpu.VMEM((CHUNK, 8), _f32),
            pltpu.VMEM((NCHUNK, CHUNK), jnp.int32),
            pltpu.VMEM_SHARED((N, 8), _f32),
            pltpu.SemaphoreType.DMA,
        ],
    )


def _tc_pre(degp, x, w0):
    def body(degp_ref, x_ref, w_ref, dinv_ref, y_ref):
        deg = degp_ref[pl.ds(0, N), :] + degp_ref[pl.ds(N, N), :] + 1.0
        dinv = lax.rsqrt(deg)
        dinv_ref[...] = dinv
        h = jnp.dot(x_ref[...], w_ref[...], preferred_element_type=_f32)
        y_ref[...] = h * dinv[:, 0:1]

    return pl.pallas_call(
        body,
        out_shape=(jax.ShapeDtypeStruct((N, 8), _f32),
                   jax.ShapeDtypeStruct((N, w0.shape[1]), _f32)),
    )(degp, x, w0)


@functools.cache
def _tc_mid(f_in, f_out, act):
    def body(accp_ref, y_ref, dinv_ref, b_ref, w_ref, ynew_ref):
        dinv = dinv_ref[...][:, 0:1]
        a = accp_ref[pl.ds(0, N), :] + accp_ref[pl.ds(N, N), :] + y_ref[...]
        z = a * dinv + b_ref[...]
        z = jnp.tanh(z) if act == "tanh" else jnp.maximum(z, 0.0)
        h = jnp.dot(z, w_ref[...], preferred_element_type=_f32)
        ynew_ref[...] = h * dinv

    return pl.pallas_call(
        body,
        out_shape=jax.ShapeDtypeStruct((N, f_out), _f32),
    )


RB = 400
NBLK = N // RB


def _tc_last(accp, y, dinv, batch_col, b3, wout, bout):
    F = 32

    def body(acc0_ref, acc1_ref, y_ref, dinv_ref, bat_ref, b_ref, wout_ref,
             bout_ref, out_ref, gmp_ref, ssum_ref, cnt_ref):
        i = pl.program_id(0)

        @pl.when(i == 0)
        def _():
            gmp_ref[...] = jnp.zeros_like(gmp_ref)
            ssum_ref[...] = jnp.zeros_like(ssum_ref)
            cnt_ref[...] = jnp.zeros_like(cnt_ref)

        a = acc0_ref[...] + acc1_ref[...] + y_ref[...]
        z = a * dinv_ref[...][:, 0:1] + b_ref[...]
        z = jnp.maximum(z, 0.0)
        batc = bat_ref[...][:, 0:1]
        seg_ids = lax.broadcasted_iota(jnp.int32, (1, B), 1)
        mask = (batc == seg_ids).astype(_f32)

        dn = (((0,), (0,)), ((), ()))
        ssum_ref[...] += lax.dot_general(mask, z, dn,
                                         preferred_element_type=_f32)
        cnt_ref[...] += lax.dot_general(mask, jnp.ones((RB, 1), _f32), dn,
                                        preferred_element_type=_f32)

        first = bat_ref[0, 0]
        last = bat_ref[RB - 1, 0]

        def seg(bidx, carry):
            m = batc == bidx
            zm = jnp.where(m, z, 0.0)
            vmax = jnp.max(zm, axis=0, keepdims=True)
            cur = gmp_ref[pl.ds(bidx, 1), :]
            gmp_ref[pl.ds(bidx, 1), :] = jnp.maximum(cur, vmax)
            return carry

        lax.fori_loop(first, last + 1, seg, 0)

        @pl.when(i == NBLK - 1)
        def _():
            cnt = cnt_ref[...]
            gm = jnp.where(cnt > 0.0, gmp_ref[...], -jnp.inf)
            gap = ssum_ref[...] / jnp.maximum(cnt, 1.0)
            w_max = wout_ref[pl.ds(0, F), :]
            w_avg = wout_ref[pl.ds(F, F), :]
            o = jnp.dot(gm, w_max, preferred_element_type=_f32)
            o += jnp.dot(gap, w_avg, preferred_element_type=_f32)
            out_ref[...] = o + bout_ref[...]

    return pl.pallas_call(
        body,
        grid=(NBLK,),
        in_specs=[
            pl.BlockSpec((RB, F), lambda i: (i, 0)),
            pl.BlockSpec((RB, F), lambda i: (NBLK + i, 0)),
            pl.BlockSpec((RB, F), lambda i: (i, 0)),
            pl.BlockSpec((RB, 8), lambda i: (i, 0)),
            pl.BlockSpec((RB, 8), lambda i: (i, 0)),
            pl.BlockSpec((1, F), lambda i: (0, 0)),
            pl.BlockSpec((2 * F, 10), lambda i: (0, 0)),
            pl.BlockSpec((1, 10), lambda i: (0, 0)),
        ],
        out_specs=pl.BlockSpec((B, 10), lambda i: (0, 0)),
        out_shape=jax.ShapeDtypeStruct((B, 10), _f32),
        scratch_shapes=[
            pltpu.VMEM((B, F), _f32),
            pltpu.VMEM((B, F), _f32),
            pltpu.VMEM((B, 1), _f32),
        ],
    )(accp, accp, y, dinv, batch_col, b3, wout, bout)


def kernel(x, edge_index, batch_index, W0, b0, W1, b1, W2, b2, W3, b3,
           Wout, bout):
    e4 = edge_index.reshape(2, NW, NCHUNK, CHUNK)
    zeros64 = jnp.zeros((N, 64), _f32)
    zeros32 = jnp.zeros((N, 32), _f32)
    zeros8 = jnp.zeros((N, 8), _f32)
    ones_chunk = jnp.ones((CHUNK, 8), _f32)
    batch8 = jnp.broadcast_to(batch_index.reshape(N, 1), (N, 8))

    degp = _sc_degree()(e4, ones_chunk, zeros8)
    dinv, y0 = _tc_pre(degp, x, W0)

    acc0 = _sc_scatter(64)(y0, e4, zeros64)
    y1 = _tc_mid(64, 64, "tanh")(acc0, y0, dinv, b0.reshape(1, -1), W1)

    acc1 = _sc_scatter(64)(y1, e4, zeros64)
    y2 = _tc_mid(64, 32, "relu")(acc1, y1, dinv, b1.reshape(1, -1), W2)

    acc2 = _sc_scatter(32)(y2, e4, zeros32)
    y3 = _tc_mid(32, 32, "relu")(acc2, y2, dinv, b2.reshape(1, -1), W3)

    acc3 = _sc_scatter(32)(y3, e4, zeros32)
    out = _tc_last(acc3, y3, dinv, batch8, b3.reshape(1, -1), Wout,
                   bout.reshape(1, -1))
    return out
